# baseline (speedup 1.0000x reference)
import functools

import jax
import jax.numpy as jnp
from jax import lax
from jax.experimental import pallas as pl
from jax.experimental.pallas import tpu as pltpu
from jax.experimental.pallas import tpu_sc as plsc

_NUM_CORES = 2
_NUM_SUBCORES = 16
_NUM_WORKERS = _NUM_CORES * _NUM_SUBCORES
_CB = 8
_LANES = 128
_HPAD = 56
_SEL_BATCH = 64


def _select_body(pairs_ref, out_ref):
    _, hist, d = out_ref.shape
    out_ref[...] = pairs_ref[:, :hist, :d]


def kernel(x, table):
    batch, hist = x.shape
    vocab, d_model = table.shape
    idxp = jnp.pad(x.astype(jnp.int32), ((0, 0), (0, _HPAD - hist)))
    idxp = idxp.reshape(batch * _HPAD)
    table_p = jnp.pad(table, ((0, 0), (0, _LANES - d_model)))

    chunk = _CB * _HPAD
    batches_per_worker = batch // _NUM_WORKERS
    rows_per_worker = batches_per_worker * _HPAD
    n_chunks = batches_per_worker // _CB
    assert n_chunks % 2 == 0 and n_chunks * chunk == rows_per_worker
    mesh = plsc.VectorSubcoreMesh(core_axis_name="c", subcore_axis_name="s")

    @functools.partial(
        pl.kernel,
        mesh=mesh,
        out_type=jax.ShapeDtypeStruct((batch, _HPAD, _LANES), table.dtype),
        scratch_types=[
            pltpu.VMEM((chunk,), jnp.int32),
            pltpu.VMEM((chunk,), jnp.int32),
            pltpu.VMEM((chunk, _LANES), jnp.float32),
            pltpu.VMEM((chunk, _LANES), jnp.float32),
            pltpu.SemaphoreType.DMA,
            pltpu.SemaphoreType.DMA,
            pltpu.SemaphoreType.DMA,
            pltpu.SemaphoreType.DMA,
        ],
    )
    def gather_k(table_hbm, idx_hbm, out_hbm, iv_a, iv_b, rows_a, rows_b,
                 g_a, g_b, w_a, w_b):
        wid = lax.axis_index("s") * _NUM_CORES + lax.axis_index("c")
        base = wid * rows_per_worker
        b_base = wid * batches_per_worker
        last = base + (n_chunks - 1) * chunk

        def load_idx(off, iv):
            pltpu.sync_copy(idx_hbm.at[pl.ds(off, chunk)], iv)

        def out_block(bj):
            return out_hbm.at[pl.ds(b_base + bj * _CB, _CB), :, :]

        load_idx(base, iv_a)
        pltpu.async_copy(table_hbm.at[iv_a], rows_a, g_a)
        load_idx(base + chunk, iv_b)
        pltpu.async_copy(table_hbm.at[iv_b], rows_b, g_b)

        @pl.loop(0, n_chunks, step=2)
        def _(j):
            off_a = base + j * chunk
            off_b = off_a + chunk
            dst_a = out_block(j)
            dst_b = out_block(j + 1)
            pltpu.make_async_copy(table_hbm.at[iv_a], rows_a, g_a).wait()
            pltpu.async_copy(rows_a.reshape(_CB, _HPAD, _LANES), dst_a, w_a)
            pltpu.make_async_copy(table_hbm.at[iv_b], rows_b, g_b).wait()
            pltpu.async_copy(rows_b.reshape(_CB, _HPAD, _LANES), dst_b, w_b)
            off_a2 = jnp.minimum(off_a + 2 * chunk, last)
            off_b2 = jnp.minimum(off_b + 2 * chunk, last)
            load_idx(off_a2, iv_a)
            pltpu.make_async_copy(rows_a.reshape(_CB, _HPAD, _LANES), dst_a,
                                  w_a).wait()
            pltpu.async_copy(table_hbm.at[iv_a], rows_a, g_a)
            load_idx(off_b2, iv_b)
            pltpu.make_async_copy(rows_b.reshape(_CB, _HPAD, _LANES), dst_b,
                                  w_b).wait()
            pltpu.async_copy(table_hbm.at[iv_b], rows_b, g_b)

        pltpu.make_async_copy(table_hbm.at[iv_a], rows_a, g_a).wait()
        pltpu.make_async_copy(table_hbm.at[iv_b], rows_b, g_b).wait()

    pairs3 = gather_k(table_p, idxp)

    out = pl.pallas_call(
        _select_body,
        grid=(batch // _SEL_BATCH,),
        in_specs=[
            pl.BlockSpec((_SEL_BATCH, _HPAD, _LANES), lambda i: (i, 0, 0)),
        ],
        out_specs=pl.BlockSpec((_SEL_BATCH, hist, d_model), lambda i: (i, 0, 0)),
        out_shape=jax.ShapeDtypeStruct((batch, hist, d_model), table.dtype),
    )(pairs3)

    return out


# speedup vs baseline: 3.3426x; 3.3426x over previous
import functools

import jax
import jax.numpy as jnp
from jax import lax
from jax.experimental import pallas as pl
from jax.experimental.pallas import tpu as pltpu
from jax.experimental.pallas import tpu_sc as plsc

_NUM_CORES = 2
_NUM_SUBCORES = 16
_NUM_WORKERS = _NUM_CORES * _NUM_SUBCORES
_CB = 8
_LANES = 128
_PAD_ROWS = 2000
_SEL_BATCH = 64


def _pad_body(t_ref, out_ref):
    d = t_ref.shape[-1]
    out_ref[:, :d] = t_ref[...]
    out_ref[:, d:] = jnp.zeros_like(out_ref[:, d:])


def _slice_body(pairs_ref, out_ref):
    d = out_ref.shape[-1]
    out_ref[...] = pairs_ref[:, :, :d]


def kernel(x, table):
    batch, hist = x.shape
    vocab, d_model = table.shape
    n = batch * hist
    idx = x.reshape(n).astype(jnp.int32)

    chunk = _CB * hist
    batches_per_worker = batch // _NUM_WORKERS
    rows_per_worker = batches_per_worker * hist
    n_chunks = batches_per_worker // _CB
    assert n_chunks % 2 == 0 and n_chunks * chunk == rows_per_worker

    table_w = pl.pallas_call(
        _pad_body,
        grid=(vocab // _PAD_ROWS,),
        in_specs=[pl.BlockSpec((_PAD_ROWS, d_model), lambda i: (i, 0))],
        out_specs=pl.BlockSpec((_PAD_ROWS, _LANES), lambda i: (i, 0)),
        out_shape=jax.ShapeDtypeStruct((vocab, _LANES), table.dtype),
    )(table)

    mesh = plsc.VectorSubcoreMesh(core_axis_name="c", subcore_axis_name="s")

    @functools.partial(
        pl.kernel,
        mesh=mesh,
        out_type=jax.ShapeDtypeStruct((batch, hist, _LANES), table.dtype),
        scratch_types=[
            pltpu.VMEM((chunk,), jnp.int32),
            pltpu.VMEM((chunk,), jnp.int32),
            pltpu.VMEM((chunk, _LANES), jnp.float32),
            pltpu.VMEM((chunk, _LANES), jnp.float32),
            pltpu.SemaphoreType.DMA,
            pltpu.SemaphoreType.DMA,
            pltpu.SemaphoreType.DMA,
            pltpu.SemaphoreType.DMA,
        ],
    )
    def gather_k(table_hbm, idx_hbm, out_hbm, iv_a, iv_b, rows_a, rows_b,
                 g_a, g_b, w_a, w_b):
        wid = lax.axis_index("s") * _NUM_CORES + lax.axis_index("c")
        base = wid * rows_per_worker
        b_base = wid * batches_per_worker
        last = base + (n_chunks - 1) * chunk

        def load_idx(off, iv):
            pltpu.sync_copy(idx_hbm.at[pl.ds(off, chunk)], iv)

        def out_block(bj):
            return out_hbm.at[pl.ds(b_base + bj * _CB, _CB)]

        load_idx(base, iv_a)
        pltpu.async_copy(table_hbm.at[iv_a], rows_a, g_a)
        load_idx(base + chunk, iv_b)
        pltpu.async_copy(table_hbm.at[iv_b], rows_b, g_b)

        @pl.loop(0, n_chunks, step=2)
        def _(j):
            off_a = base + j * chunk
            off_b = off_a + chunk
            dst_a = out_block(j)
            dst_b = out_block(j + 1)
            pltpu.make_async_copy(table_hbm.at[iv_a], rows_a, g_a).wait()
            pltpu.async_copy(rows_a.reshape(_CB, hist, _LANES), dst_a, w_a)
            pltpu.make_async_copy(table_hbm.at[iv_b], rows_b, g_b).wait()
            pltpu.async_copy(rows_b.reshape(_CB, hist, _LANES), dst_b, w_b)
            off_a2 = jnp.minimum(off_a + 2 * chunk, last)
            off_b2 = jnp.minimum(off_b + 2 * chunk, last)
            load_idx(off_a2, iv_a)
            pltpu.make_async_copy(rows_a.reshape(_CB, hist, _LANES), dst_a,
                                  w_a).wait()
            pltpu.async_copy(table_hbm.at[iv_a], rows_a, g_a)
            load_idx(off_b2, iv_b)
            pltpu.make_async_copy(rows_b.reshape(_CB, hist, _LANES), dst_b,
                                  w_b).wait()
            pltpu.async_copy(table_hbm.at[iv_b], rows_b, g_b)

        pltpu.make_async_copy(table_hbm.at[iv_a], rows_a, g_a).wait()
        pltpu.make_async_copy(table_hbm.at[iv_b], rows_b, g_b).wait()

    pairs3 = gather_k(table_w, idx)

    out = pl.pallas_call(
        _slice_body,
        grid=(batch // _SEL_BATCH,),
        in_specs=[
            pl.BlockSpec((_SEL_BATCH, hist, _LANES), lambda i: (i, 0, 0)),
        ],
        out_specs=pl.BlockSpec((_SEL_BATCH, hist, d_model), lambda i: (i, 0, 0)),
        out_shape=jax.ShapeDtypeStruct((batch, hist, d_model), table.dtype),
    )(pairs3)

    return out


# speedup vs baseline: 3.7444x; 1.1202x over previous
import functools

import jax
import jax.numpy as jnp
from jax import lax
from jax.experimental import pallas as pl
from jax.experimental.pallas import tpu as pltpu
from jax.experimental.pallas import tpu_sc as plsc

_NUM_CORES = 2
_NUM_SUBCORES = 16
_NUM_WORKERS = _NUM_CORES * _NUM_SUBCORES
_CB = 8
_LANES = 128
_PAD_ROWS = 4000
_SEL_BATCH = 256


def _pad_body(t_ref, out_ref):
    d = t_ref.shape[-1]
    out_ref[:, :d] = t_ref[...]


def _slice_body(pairs_ref, out_ref):
    d = out_ref.shape[-1]
    out_ref[...] = pairs_ref[:, :, :d]


def kernel(x, table):
    batch, hist = x.shape
    vocab, d_model = table.shape
    n = batch * hist
    idx = x.reshape(n).astype(jnp.int32)

    chunk = _CB * hist
    batches_per_worker = batch // _NUM_WORKERS
    rows_per_worker = batches_per_worker * hist
    n_chunks = batches_per_worker // _CB
    assert n_chunks % 2 == 0 and n_chunks * chunk == rows_per_worker

    table_w = pl.pallas_call(
        _pad_body,
        grid=(vocab // _PAD_ROWS,),
        in_specs=[pl.BlockSpec((_PAD_ROWS, d_model), lambda i: (i, 0))],
        out_specs=pl.BlockSpec((_PAD_ROWS, _LANES), lambda i: (i, 0)),
        out_shape=jax.ShapeDtypeStruct((vocab, _LANES), table.dtype),
    )(table)

    mesh = plsc.VectorSubcoreMesh(core_axis_name="c", subcore_axis_name="s")

    @functools.partial(
        pl.kernel,
        mesh=mesh,
        out_type=jax.ShapeDtypeStruct((batch, hist, _LANES), table.dtype),
        scratch_types=[
            pltpu.VMEM((chunk,), jnp.int32),
            pltpu.VMEM((chunk,), jnp.int32),
            pltpu.VMEM((chunk, _LANES), jnp.float32),
            pltpu.VMEM((chunk, _LANES), jnp.float32),
            pltpu.SemaphoreType.DMA,
            pltpu.SemaphoreType.DMA,
            pltpu.SemaphoreType.DMA,
            pltpu.SemaphoreType.DMA,
        ],
    )
    def gather_k(table_hbm, idx_hbm, out_hbm, iv_a, iv_b, rows_a, rows_b,
                 g_a, g_b, w_a, w_b):
        wid = lax.axis_index("s") * _NUM_CORES + lax.axis_index("c")
        base = wid * rows_per_worker
        b_base = wid * batches_per_worker
        last = base + (n_chunks - 1) * chunk

        def load_idx(off, iv):
            pltpu.sync_copy(idx_hbm.at[pl.ds(off, chunk)], iv)

        def out_block(bj):
            return out_hbm.at[pl.ds(b_base + bj * _CB, _CB)]

        load_idx(base, iv_a)
        pltpu.async_copy(table_hbm.at[iv_a], rows_a, g_a)
        load_idx(base + chunk, iv_b)
        pltpu.async_copy(table_hbm.at[iv_b], rows_b, g_b)

        @pl.loop(0, n_chunks, step=2)
        def _(j):
            off_a = base + j * chunk
            off_b = off_a + chunk
            dst_a = out_block(j)
            dst_b = out_block(j + 1)
            pltpu.make_async_copy(table_hbm.at[iv_a], rows_a, g_a).wait()
            pltpu.async_copy(rows_a.reshape(_CB, hist, _LANES), dst_a, w_a)
            pltpu.make_async_copy(table_hbm.at[iv_b], rows_b, g_b).wait()
            pltpu.async_copy(rows_b.reshape(_CB, hist, _LANES), dst_b, w_b)
            off_a2 = jnp.minimum(off_a + 2 * chunk, last)
            off_b2 = jnp.minimum(off_b + 2 * chunk, last)
            load_idx(off_a2, iv_a)
            pltpu.make_async_copy(rows_a.reshape(_CB, hist, _LANES), dst_a,
                                  w_a).wait()
            pltpu.async_copy(table_hbm.at[iv_a], rows_a, g_a)
            load_idx(off_b2, iv_b)
            pltpu.make_async_copy(rows_b.reshape(_CB, hist, _LANES), dst_b,
                                  w_b).wait()
            pltpu.async_copy(table_hbm.at[iv_b], rows_b, g_b)

        pltpu.make_async_copy(table_hbm.at[iv_a], rows_a, g_a).wait()
        pltpu.make_async_copy(table_hbm.at[iv_b], rows_b, g_b).wait()

    pairs3 = gather_k(table_w, idx)

    out = pl.pallas_call(
        _slice_body,
        grid=(batch // _SEL_BATCH,),
        in_specs=[
            pl.BlockSpec((_SEL_BATCH, hist, _LANES), lambda i: (i, 0, 0)),
        ],
        out_specs=pl.BlockSpec((_SEL_BATCH, hist, d_model), lambda i: (i, 0, 0)),
        out_shape=jax.ShapeDtypeStruct((batch, hist, d_model), table.dtype),
    )(pairs3)

    return out


# speedup vs baseline: 4.5596x; 1.2177x over previous
import functools

import jax
import jax.numpy as jnp
from jax import lax
from jax.experimental import pallas as pl
from jax.experimental.pallas import tpu as pltpu
from jax.experimental.pallas import tpu_sc as plsc

_NUM_CORES = 2
_NUM_SUBCORES = 16
_NUM_WORKERS = _NUM_CORES * _NUM_SUBCORES
_CB = 8
_LANES = 128
_PAD_ROWS = 4000
_SEL_BATCH = 256


def _pad_body(t_ref, out_ref):
    d = t_ref.shape[-1]
    out_ref[:, :d] = t_ref[...]


def _slice_body(pairs_ref, out_ref):
    d = out_ref.shape[1]
    out_ref[...] = jnp.transpose(pairs_ref[:, :, :d], (1, 2, 0))


def kernel(x, table):
    batch, hist = x.shape
    vocab, d_model = table.shape
    n = batch * hist
    idx = x.reshape(n).astype(jnp.int32)

    chunk = _CB * hist
    batches_per_worker = batch // _NUM_WORKERS
    rows_per_worker = batches_per_worker * hist
    n_chunks = batches_per_worker // _CB
    assert n_chunks % 2 == 0 and n_chunks * chunk == rows_per_worker

    table_w = pl.pallas_call(
        _pad_body,
        grid=(vocab // _PAD_ROWS,),
        in_specs=[pl.BlockSpec((_PAD_ROWS, d_model), lambda i: (i, 0))],
        out_specs=pl.BlockSpec((_PAD_ROWS, _LANES), lambda i: (i, 0)),
        out_shape=jax.ShapeDtypeStruct((vocab, _LANES), table.dtype),
    )(table)

    mesh = plsc.VectorSubcoreMesh(core_axis_name="c", subcore_axis_name="s")

    @functools.partial(
        pl.kernel,
        mesh=mesh,
        out_type=jax.ShapeDtypeStruct((batch, hist, _LANES), table.dtype),
        scratch_types=[
            pltpu.VMEM((chunk,), jnp.int32),
            pltpu.VMEM((chunk,), jnp.int32),
            pltpu.VMEM((chunk, _LANES), jnp.float32),
            pltpu.VMEM((chunk, _LANES), jnp.float32),
            pltpu.SemaphoreType.DMA,
            pltpu.SemaphoreType.DMA,
            pltpu.SemaphoreType.DMA,
            pltpu.SemaphoreType.DMA,
        ],
    )
    def gather_k(table_hbm, idx_hbm, out_hbm, iv_a, iv_b, rows_a, rows_b,
                 g_a, g_b, w_a, w_b):
        wid = lax.axis_index("s") * _NUM_CORES + lax.axis_index("c")
        base = wid * rows_per_worker
        b_base = wid * batches_per_worker
        last = base + (n_chunks - 1) * chunk

        def load_idx(off, iv):
            pltpu.sync_copy(idx_hbm.at[pl.ds(off, chunk)], iv)

        def out_block(bj):
            return out_hbm.at[pl.ds(b_base + bj * _CB, _CB)]

        load_idx(base, iv_a)
        pltpu.async_copy(table_hbm.at[iv_a], rows_a, g_a)
        load_idx(base + chunk, iv_b)
        pltpu.async_copy(table_hbm.at[iv_b], rows_b, g_b)

        @pl.loop(0, n_chunks, step=2)
        def _(j):
            off_a = base + j * chunk
            off_b = off_a + chunk
            dst_a = out_block(j)
            dst_b = out_block(j + 1)
            pltpu.make_async_copy(table_hbm.at[iv_a], rows_a, g_a).wait()
            pltpu.async_copy(rows_a.reshape(_CB, hist, _LANES), dst_a, w_a)
            pltpu.make_async_copy(table_hbm.at[iv_b], rows_b, g_b).wait()
            pltpu.async_copy(rows_b.reshape(_CB, hist, _LANES), dst_b, w_b)
            off_a2 = jnp.minimum(off_a + 2 * chunk, last)
            off_b2 = jnp.minimum(off_b + 2 * chunk, last)
            load_idx(off_a2, iv_a)
            pltpu.make_async_copy(rows_a.reshape(_CB, hist, _LANES), dst_a,
                                  w_a).wait()
            pltpu.async_copy(table_hbm.at[iv_a], rows_a, g_a)
            load_idx(off_b2, iv_b)
            pltpu.make_async_copy(rows_b.reshape(_CB, hist, _LANES), dst_b,
                                  w_b).wait()
            pltpu.async_copy(table_hbm.at[iv_b], rows_b, g_b)

        pltpu.make_async_copy(table_hbm.at[iv_a], rows_a, g_a).wait()
        pltpu.make_async_copy(table_hbm.at[iv_b], rows_b, g_b).wait()

    pairs3 = gather_k(table_w, idx)

    out_t = pl.pallas_call(
        _slice_body,
        grid=(batch // _SEL_BATCH,),
        in_specs=[
            pl.BlockSpec((_SEL_BATCH, hist, _LANES), lambda i: (i, 0, 0)),
        ],
        out_specs=pl.BlockSpec((hist, d_model, _SEL_BATCH), lambda i: (0, 0, i)),
        out_shape=jax.ShapeDtypeStruct((hist, d_model, batch), table.dtype),
    )(pairs3)

    return out_t.transpose(2, 0, 1)
